# Initial kernel scaffold; baseline (speedup 1.0000x reference)
import jax
import jax.numpy as jnp
from jax.experimental import pallas as pl


def kernel(x, emb, pos_emb):
    raise NotImplementedError("write your pallas kernel here")


# speedup vs baseline: 2.0572x; 2.0572x over previous
import functools

import jax
import jax.numpy as jnp
from jax import lax
from jax.experimental import pallas as pl
from jax.experimental.pallas import tpu as pltpu
from jax.experimental.pallas import tpu_sc as plsc

_NUM_HID = 128
_POS_ROWS = 200
_CHUNK = 128


@functools.cache
def _build(total, hid, pos_rows):
    info = plsc.get_sparse_core_info()
    nc, ns = info.num_cores, info.num_subcores
    nw = nc * ns
    b_per_w = total // nw
    n_chunks = b_per_w // _CHUNK
    mesh = plsc.VectorSubcoreMesh(core_axis_name="c", subcore_axis_name="s")

    @functools.partial(
        pl.kernel,
        out_type=jax.ShapeDtypeStruct((total, hid), jnp.float32),
        mesh=mesh,
        scratch_types=[
            pltpu.VMEM((b_per_w,), jnp.int32),
            pltpu.VMEM((pos_rows, hid), jnp.float32),
            pltpu.VMEM((_CHUNK, hid), jnp.float32),
            pltpu.SemaphoreType.DMA,
        ],
    )
    def emb_kernel(x_hbm, emb_hbm, pos_hbm, out_hbm, idx_v, pos_v, rows_v, sem):
        wid = lax.axis_index("s") * nc + lax.axis_index("c")
        base = wid * b_per_w
        pltpu.sync_copy(x_hbm.at[pl.ds(base, b_per_w)], idx_v)
        pltpu.sync_copy(pos_hbm, pos_v)

        def chunk_body(j, carry):
            pltpu.async_copy(
                emb_hbm.at[idx_v.at[pl.ds(j * _CHUNK, _CHUNK)]], rows_v, sem
            ).wait()

            def row_body(r, c2):
                p = lax.rem(j * _CHUNK + r, pos_rows)
                for c in range(hid // 16):
                    sl = pl.ds(c * 16, 16)
                    rows_v[r, sl] = rows_v[r, sl] + pos_v[p, sl]
                return c2

            lax.fori_loop(0, _CHUNK, row_body, 0)
            pltpu.sync_copy(rows_v, out_hbm.at[pl.ds(base + j * _CHUNK, _CHUNK)])
            return carry

        lax.fori_loop(0, n_chunks, chunk_body, 0)

    return emb_kernel


def kernel(x, emb, pos_emb):
    batch, seq = x.shape
    x_flat = x.reshape(-1).astype(jnp.int32)
    out = _build(batch * seq, emb.shape[1], pos_emb.shape[0])(x_flat, emb, pos_emb)
    return out.reshape(batch, seq, emb.shape[1])


# speedup vs baseline: 6.8571x; 3.3331x over previous
import functools

import jax
import jax.numpy as jnp
from jax import lax
from jax.experimental import pallas as pl
from jax.experimental.pallas import tpu as pltpu
from jax.experimental.pallas import tpu_sc as plsc

_SEQ = 200
_SPLIT = 104
_NBUF = 3


@functools.cache
def _build(total, hid, pos_rows):
    info = plsc.get_sparse_core_info()
    nc, ns = info.num_cores, info.num_subcores
    nw = nc * ns
    b_per_w = total // nw
    n_chunks = b_per_w // pos_rows
    lanes = hid // 16
    mesh = plsc.VectorSubcoreMesh(core_axis_name="c", subcore_axis_name="s")

    @functools.partial(
        pl.kernel,
        out_type=jax.ShapeDtypeStruct((total, hid), jnp.float32),
        mesh=mesh,
        scratch_types=[
            pltpu.VMEM((b_per_w,), jnp.int32),
            pltpu.VMEM((pos_rows, hid), jnp.float32),
            [pltpu.VMEM((pos_rows, hid), jnp.float32) for _ in range(_NBUF)],
            [pltpu.SemaphoreType.DMA for _ in range(_NBUF)],
            [pltpu.SemaphoreType.DMA for _ in range(_NBUF)],
        ],
    )
    def emb_kernel(x_hbm, emb_hbm, pos_hbm, out_hbm, idx_v, pos_v, bufs, gsems, ssems):
        wid = lax.axis_index("s") * nc + lax.axis_index("c")
        base = wid * b_per_w
        pltpu.sync_copy(x_hbm.at[pl.ds(base, b_per_w)], idx_v)
        pltpu.sync_copy(pos_hbm, pos_v)

        def issue_gather(j):
            b = j % _NBUF
            off = j * pos_rows
            d1 = pltpu.async_copy(
                emb_hbm.at[idx_v.at[pl.ds(off, _SPLIT)]],
                bufs[b].at[pl.ds(0, _SPLIT)],
                gsems[b],
            )
            d2 = pltpu.async_copy(
                emb_hbm.at[idx_v.at[pl.ds(off + _SPLIT, pos_rows - _SPLIT)]],
                bufs[b].at[pl.ds(_SPLIT, pos_rows - _SPLIT)],
                gsems[b],
            )
            return d1, d2

        def add_pos(buf):
            def row(r):
                for c in range(lanes):
                    sl = pl.ds(c * 16, 16)
                    plsc.addupdate(buf.at[r, sl], pos_v[r, sl])

            plsc.parallel_loop(0, pos_rows, 1, unroll=4)(row)

        g = [None] * n_chunks
        s = [None] * n_chunks
        g[0] = issue_gather(0)
        for j in range(n_chunks):
            b = j % _NBUF
            if j >= 2:
                s[j - 2].wait()
            if j + 1 < n_chunks:
                g[j + 1] = issue_gather(j + 1)
            g[j][0].wait()
            g[j][1].wait()
            add_pos(bufs[b])
            s[j] = pltpu.async_copy(
                bufs[b], out_hbm.at[pl.ds(base + j * pos_rows, pos_rows)], ssems[b]
            )
        s[n_chunks - 2].wait()
        s[n_chunks - 1].wait()

    return emb_kernel


def kernel(x, emb, pos_emb):
    batch, seq = x.shape
    x_flat = x.reshape(-1).astype(jnp.int32)
    out = _build(batch * seq, emb.shape[1], pos_emb.shape[0])(x_flat, emb, pos_emb)
    return out.reshape(batch, seq, emb.shape[1])


# speedup vs baseline: 6.9834x; 1.0184x over previous
import functools

import jax
import jax.numpy as jnp
from jax import lax
from jax.experimental import pallas as pl
from jax.experimental.pallas import tpu as pltpu
from jax.experimental.pallas import tpu_sc as plsc

_SPLIT = 104
_NBUF = 6


@functools.cache
def _build(total, hid, pos_rows):
    info = plsc.get_sparse_core_info()
    nc, ns = info.num_cores, info.num_subcores
    nw = nc * ns
    b_per_w = total // nw
    n_sub = 2 * (b_per_w // pos_rows)
    lanes = hid // 16
    mesh = plsc.VectorSubcoreMesh(core_axis_name="c", subcore_axis_name="s")

    def sub_off_len(k):
        off = (k // 2) * pos_rows + (k % 2) * _SPLIT
        ln = _SPLIT if k % 2 == 0 else pos_rows - _SPLIT
        return off, ln, (k % 2) * _SPLIT

    @functools.partial(
        pl.kernel,
        out_type=jax.ShapeDtypeStruct((total, hid), jnp.float32),
        mesh=mesh,
        scratch_types=[
            pltpu.VMEM((b_per_w,), jnp.int32),
            pltpu.VMEM((pos_rows, hid), jnp.float32),
            [pltpu.VMEM((_SPLIT, hid), jnp.float32) for _ in range(_NBUF)],
            [pltpu.SemaphoreType.DMA for _ in range(_NBUF)],
            [pltpu.SemaphoreType.DMA for _ in range(_NBUF)],
        ],
    )
    def emb_kernel(x_hbm, emb_hbm, pos_hbm, out_hbm, idx_v, pos_v, bufs, gsems, ssems):
        wid = lax.axis_index("s") * nc + lax.axis_index("c")
        base = wid * b_per_w
        pltpu.sync_copy(x_hbm.at[pl.ds(base, b_per_w)], idx_v)
        pltpu.sync_copy(pos_hbm, pos_v)

        def issue_gather(k):
            off, ln, _ = sub_off_len(k)
            b = k % _NBUF
            return pltpu.async_copy(
                emb_hbm.at[idx_v.at[pl.ds(off, ln)]],
                bufs[b].at[pl.ds(0, ln)],
                gsems[b],
            )

        def add_pos(k):
            _, ln, po = sub_off_len(k)
            buf = bufs[k % _NBUF]

            def row(i, carry):
                for u in range(2):
                    r = i * 2 + u
                    for c in range(lanes):
                        sl = pl.ds(c * 16, 16)
                        plsc.addupdate(buf.at[r, sl], pos_v[po + r, sl])
                return carry

            lax.fori_loop(0, ln // 2, row, 0)

        def issue_store(k):
            off, ln, _ = sub_off_len(k)
            b = k % _NBUF
            return pltpu.async_copy(
                bufs[b].at[pl.ds(0, ln)],
                out_hbm.at[pl.ds(base + off, ln)],
                ssems[b],
            )

        g = [None] * n_sub
        s = [None] * n_sub
        g[0] = issue_gather(0)
        g[1] = issue_gather(1)
        for k in range(n_sub):
            if k >= _NBUF - 2:
                s[k - (_NBUF - 2)].wait()
            if k + 2 < n_sub:
                g[k + 2] = issue_gather(k + 2)
            g[k].wait()
            add_pos(k)
            s[k] = issue_store(k)
        for k in range(n_sub - (_NBUF - 2), n_sub):
            s[k].wait()

    return emb_kernel


def kernel(x, emb, pos_emb):
    batch, seq = x.shape
    x_flat = x.reshape(-1).astype(jnp.int32)
    out = _build(batch * seq, emb.shape[1], pos_emb.shape[0])(x_flat, emb, pos_emb)
    return out.reshape(batch, seq, emb.shape[1])
